# Initial kernel scaffold; baseline (speedup 1.0000x reference)
import jax
import jax.numpy as jnp
from jax.experimental import pallas as pl


def kernel(edge_index, edge_vals, uEmbeds0, iEmbeds0, K, Vp, Vt, uHyper, iHyper, W1_0, W2_0, W1_1, W2_1):
    raise NotImplementedError("write your pallas kernel here")


# speedup vs baseline: 1.2604x; 1.2604x over previous
import functools
import jax
import jax.numpy as jnp
from jax import lax
from jax.experimental import pallas as pl
from jax.experimental.pallas import tpu as pltpu
from jax.experimental.pallas import tpu_sc as plsc

N_NODES = 50000
LATDIM = 64
HYPERNUM = 128
HEADS = 4
DH = LATDIM // HEADS
LEAKY = 0.5

E_TOTAL = 800000
N_TILES = 16
N_CORES = 2
EDGES_PER_TILE = E_TOTAL // N_TILES
CHUNK = 80
N_CHUNKS = EDGES_PER_TILE // CHUNK
HALF = N_NODES // N_CORES
ACC_ROWS = 25600
ROWS_PER_TILE = ACC_ROWS // N_TILES
DUMMY_ROW = HALF


def _spmm_body(dst_hbm, src_hbm, vals_hbm, table_hbm, out_hbm,
               di, si, vv, rows, zbuf, acc, sem):
    cid = lax.axis_index("c")
    sid = lax.axis_index("s")
    base_row = cid * HALF
    tile_row0 = sid * ROWS_PER_TILE

    for i in range(8):
        for j in range(4):
            zbuf[i, 16 * j:16 * (j + 1)] = jnp.zeros((16,), jnp.float32)

    def zero_step(k, _):
        pltpu.sync_copy(zbuf, acc.at[pl.ds(tile_row0 + 8 * k, 8)])
        return 0
    lax.fori_loop(0, ROWS_PER_TILE // 8, zero_step, 0)
    plsc.subcore_barrier()

    def chunk_step(k, _):
        eb = sid * EDGES_PER_TILE + k * CHUNK
        pltpu.sync_copy(dst_hbm.at[pl.ds(eb, CHUNK)], di.at[0])
        pltpu.sync_copy(src_hbm.at[pl.ds(eb, CHUNK)], si)
        pltpu.sync_copy(vals_hbm.at[pl.ds(eb, CHUNK)], vv)
        pltpu.async_copy(table_hbm.at[si], rows, sem).wait()
        for j in range(CHUNK // 16):
            d = di[0, 16 * j:16 * (j + 1)]
            dl = d - base_row
            ok = (dl >= 0) & (dl < HALF)
            di[0, 16 * j:16 * (j + 1)] = jnp.where(ok, dl, DUMMY_ROW)

        def scale_grp(g, _):
            sv = vv[pl.ds(g * 16, 16)]
            for l in range(16):
                i = g * 16 + l
                s = sv[l]
                for j in range(4):
                    rows[i, 16 * j:16 * (j + 1)] = (
                        rows[i, 16 * j:16 * (j + 1)] * s)
            return 0
        lax.fori_loop(0, CHUNK // 16, scale_grp, 0)

        pltpu.sync_copy(rows, acc.at[di.at[0]], add=True)
        return 0
    lax.fori_loop(0, N_CHUNKS, chunk_step, 0)
    plsc.subcore_barrier()

    def out_step(k, _):
        a = tile_row0 + 8 * k

        @pl.when(a < HALF)
        def _():
            pltpu.sync_copy(acc.at[pl.ds(a, 8)],
                            out_hbm.at[pl.ds(base_row + a, 8)])
        return 0
    lax.fori_loop(0, ROWS_PER_TILE // 8, out_step, 0)


def _spmm(dst, src, vals, table):
    mesh = plsc.VectorSubcoreMesh(core_axis_name="c", subcore_axis_name="s")
    f = functools.partial(
        pl.kernel, _spmm_body, mesh=mesh,
        compiler_params=pltpu.CompilerParams(use_tc_tiling_on_sc=False),
        out_type=jax.ShapeDtypeStruct((N_NODES, LATDIM), jnp.float32),
        scratch_types=[
            pltpu.VMEM((1, CHUNK), jnp.int32),
            pltpu.VMEM((CHUNK,), jnp.int32),
            pltpu.VMEM((CHUNK,), jnp.float32),
            pltpu.VMEM((CHUNK, LATDIM), jnp.float32),
            pltpu.VMEM((8, LATDIM), jnp.float32),
            pltpu.VMEM_SHARED((ACC_ROWS, LATDIM), jnp.float32),
            pltpu.SemaphoreType.DMA,
        ],
    )()
    return f(dst, src, vals, table)


BLK = 2000
N_BLKS = N_NODES // BLK

_row_spec = pl.BlockSpec((BLK, LATDIM), lambda i: (i, 0))


def _full_spec(shape):
    return pl.BlockSpec(shape, lambda i: tuple(0 for _ in shape))


def _keym_body(t1_ref, t2_ref, k_ref, g_ref, keym_ref):
    g = t1_ref[...] + t2_ref[...]
    g_ref[...] = g
    keym_ref[...] = jax.lax.dot_general(
        g, k_ref[...], (((1,), (0,)), ((), ())),
        preferred_element_type=jnp.float32)


def _keym(t1, t2, K):
    return pl.pallas_call(
        _keym_body,
        grid=(N_BLKS,),
        in_specs=[_row_spec, _row_spec, _full_spec((LATDIM, LATDIM))],
        out_specs=(_row_spec, _row_spec),
        out_shape=(jax.ShapeDtypeStruct((N_NODES, LATDIM), jnp.float32),
                   jax.ShapeDtypeStruct((N_NODES, LATDIM), jnp.float32)),
    )(t1, t2, K)


def _leaky(x):
    return jnp.where(x >= 0, x, LEAKY * x)


def _reduce_body(lat_ref, keym_ref, vp_ref, temas_ref):
    @pl.when(pl.program_id(0) == 0)
    def _():
        temas_ref[...] = jnp.zeros_like(temas_ref)

    latvp = jax.lax.dot_general(lat_ref[...], vp_ref[...],
                                (((1,), (0,)), ((), ())),
                                preferred_element_type=jnp.float32)
    keym = keym_ref[...]
    for h in range(HEADS):
        hs = slice(h * DH, (h + 1) * DH)
        temas_ref[hs, :] += jax.lax.dot_general(
            latvp[:, hs], keym[:, hs], (((0,), (0,)), ((), ())),
            preferred_element_type=jnp.float32)


def _mlp_body(temas_ref, hyper_ref, vt_ref, w1_ref, w2_ref, pre2_ref):
    hyper = hyper_ref[...]
    parts = []
    for h in range(HEADS):
        hs = slice(h * DH, (h + 1) * DH)
        parts.append(jax.lax.dot_general(
            temas_ref[hs, :], hyper[:, hs], (((1,), (1,)), ((), ())),
            preferred_element_type=jnp.float32))
    temlat1 = jnp.concatenate(parts, axis=0)
    temlat2 = _leaky(jax.lax.dot_general(
        temlat1, w1_ref[...], (((1,), (1,)), ((), ())),
        preferred_element_type=jnp.float32)) + temlat1
    temlat3 = _leaky(jax.lax.dot_general(
        temlat2, w2_ref[...], (((1,), (1,)), ((), ())),
        preferred_element_type=jnp.float32)) + temlat2
    prem = jax.lax.dot_general(temlat3, vt_ref[...],
                               (((0,), (0,)), ((), ())),
                               preferred_element_type=jnp.float32)
    for h in range(HEADS):
        hs = slice(h * DH, (h + 1) * DH)
        pre2_ref[hs, :] = jax.lax.dot_general(
            hyper[:, hs], prem[:, hs], (((0,), (0,)), ((), ())),
            preferred_element_type=jnp.float32)


def _bcast_body(keym_ref, pre2_ref, out_ref):
    keym = keym_ref[...]
    pre2 = pre2_ref[...]
    outs = []
    for h in range(HEADS):
        hs = slice(h * DH, (h + 1) * DH)
        outs.append(jax.lax.dot_general(keym[:, hs], pre2[hs, :],
                                        (((1,), (0,)), ((), ())),
                                        preferred_element_type=jnp.float32))
    out_ref[...] = jnp.concatenate(outs, axis=1)


def _hgt_layer(lat, keym, Vp, Vt, Hyper, W1, W2):
    temas = pl.pallas_call(
        _reduce_body,
        grid=(N_BLKS,),
        in_specs=[_row_spec, _row_spec, _full_spec((LATDIM, LATDIM))],
        out_specs=_full_spec((LATDIM, DH)),
        out_shape=jax.ShapeDtypeStruct((LATDIM, DH), jnp.float32),
    )(lat, keym, Vp)
    pre2 = pl.pallas_call(
        _mlp_body,
        out_shape=jax.ShapeDtypeStruct((LATDIM, DH), jnp.float32),
    )(temas, Hyper, Vt, W1, W2)
    return pl.pallas_call(
        _bcast_body,
        grid=(N_BLKS,),
        in_specs=[_row_spec, _full_spec((LATDIM, DH))],
        out_specs=_row_spec,
        out_shape=jax.ShapeDtypeStruct((N_NODES, LATDIM), jnp.float32),
    )(keym, pre2)


def _add3_body(a_ref, b_ref, c_ref, o_ref):
    o_ref[...] = a_ref[...] + b_ref[...] + c_ref[...]


def _add3(a, b, c):
    return pl.pallas_call(
        _add3_body,
        grid=(N_BLKS,),
        in_specs=[_row_spec, _row_spec, _row_spec],
        out_specs=_row_spec,
        out_shape=jax.ShapeDtypeStruct((N_NODES, LATDIM), jnp.float32),
    )(a, b, c)


def kernel(edge_index, edge_vals, uEmbeds0, iEmbeds0, K, Vp, Vt, uHyper,
           iHyper, W1_0, W2_0, W1_1, W2_1):
    r = edge_index[0]
    c = edge_index[1]
    temu1 = _spmm(r, c, edge_vals, iEmbeds0)
    temi1 = _spmm(c, r, edge_vals, uEmbeds0)
    temu2 = _spmm(r, c, edge_vals, temi1)
    temi2 = _spmm(c, r, edge_vals, temu1)

    uG, ukeym = _keym(temu1, temu2, K)
    iG, ikeym = _keym(temi1, temi2, K)

    u1 = _hgt_layer(uG, ukeym, Vp, Vt, uHyper, W1_0, W2_0)
    u2 = _hgt_layer(u1, ukeym, Vp, Vt, uHyper, W1_1, W2_1)
    i1 = _hgt_layer(iG, ikeym, Vp, Vt, iHyper, W1_0, W2_0)
    i2 = _hgt_layer(i1, ikeym, Vp, Vt, iHyper, W1_1, W2_1)

    uOut = _add3(uG, u1, u2)
    iOut = _add3(iG, i1, i2)
    return jnp.concatenate([uOut, iOut], axis=0)
